# Initial kernel scaffold; baseline (speedup 1.0000x reference)
import jax
import jax.numpy as jnp
from jax.experimental import pallas as pl


def kernel(input_data, FRB_weights, c1, c2):
    raise NotImplementedError("write your pallas kernel here")


# speedup vs baseline: 2.0832x; 2.0832x over previous
import jax
import jax.numpy as jnp
from jax.experimental import pallas as pl
from jax.experimental.pallas import tpu as pltpu

_R = 32
_A = 6
_BBLK = 512


def _rank_mask(c_col, c_row):
    ii = jax.lax.broadcasted_iota(jnp.int32, (_R, _R), 0)
    jj = jax.lax.broadcasted_iota(jnp.int32, (_R, _R), 1)
    cmp = (c_col < c_row) | ((c_col == c_row) & (ii < jj))
    rank = jnp.sum(cmp.astype(jnp.int32), axis=0, keepdims=True)
    pre = (rank <= ii).astype(jnp.float32)
    return pre, 1.0 - pre


def _body(xt_ref, sig_ref, ma_ref, mb_ref,
          c1c_ref, c1r_ref, c2c_ref, c2r_ref, out_ref):
    sig = sig_ref[...] + 0.0001
    m1 = jnp.minimum(ma_ref[...], mb_ref[...])
    m2 = jnp.maximum(ma_ref[...], mb_ref[...])
    inv = 1.0 / (2.0 * sig * sig)
    mid = (m1 + m2) * 0.5

    eU = jnp.zeros((_R, _BBLK), jnp.float32)
    eL = jnp.zeros((_R, _BBLK), jnp.float32)
    for a in range(_A):
        xa = xt_ref[a:a + 1, :]
        m1a = m1[:, a:a + 1]
        m2a = m2[:, a:a + 1]
        inva = inv[:, a:a + 1]
        mida = mid[:, a:a + 1]
        d1 = xa - m1a
        d2 = xa - m2a
        e1 = -(d1 * d1) * inva
        e2 = -(d2 * d2) * inva
        up = jnp.where((xa >= m1a) & (xa <= m2a), 0.0,
                       jnp.where(xa > m2a, e2, e1))
        lo = jnp.where(xa <= mida, e2, e1)
        eU = eU + up
        eL = eL + lo
    emax = jnp.max(eU, axis=0, keepdims=True)
    UU = jnp.exp(eU - emax)
    LL = jnp.exp(eL - emax)

    c1c = c1c_ref[...]
    c2c = c2c_ref[...]
    M1p, M1s = _rank_mask(c1c, c1r_ref[...])
    M2p, M2s = _rank_mask(c2c, c2r_ref[...])

    def _dot(m, v):
        return jnp.dot(m, v, preferred_element_type=jnp.float32)

    c1U = c1c * UU
    c1L = c1c * LL
    s0 = jnp.sum(c1L, axis=0, keepdims=True)
    s10 = jnp.sum(LL, axis=0, keepdims=True)
    s = _dot(M1p, c1U) + _dot(M1s, c1L)
    s1 = _dot(M1p, UU) + _dot(M1s, LL)
    left = jnp.minimum(s0 / s10, jnp.min(s / s1, axis=0, keepdims=True))

    c2U = c2c * UU
    c2L = c2c * LL
    t0 = jnp.sum(c2U, axis=0, keepdims=True)
    t10 = jnp.sum(UU, axis=0, keepdims=True)
    t = _dot(M2p, c2L) + _dot(M2s, c2U)
    t1 = _dot(M2p, LL) + _dot(M2s, UU)
    right = jnp.maximum(t0 / t10, jnp.max(t / t1, axis=0, keepdims=True))

    out_ref[...] = (left + right) * 0.5


def kernel(input_data, FRB_weights, c1, c2):
    B = input_data.shape[0]
    xt = input_data.T
    sig = FRB_weights[0:_R * _A].reshape(_R, _A)
    ma = FRB_weights[1:_R * _A + 1].reshape(_R, _A)
    mb = FRB_weights[2:_R * _A + 2].reshape(_R, _A)
    c1c = c1.reshape(_R, 1)
    c1r = c1.reshape(1, _R)
    c2c = c2.reshape(_R, 1)
    c2r = c2.reshape(1, _R)

    grid = (B // _BBLK,)
    rep = lambda i: (0, 0)
    out = pl.pallas_call(
        _body,
        grid=grid,
        in_specs=[
            pl.BlockSpec((_A, _BBLK), lambda i: (0, i)),
            pl.BlockSpec((_R, _A), rep),
            pl.BlockSpec((_R, _A), rep),
            pl.BlockSpec((_R, _A), rep),
            pl.BlockSpec((_R, 1), rep),
            pl.BlockSpec((1, _R), rep),
            pl.BlockSpec((_R, 1), rep),
            pl.BlockSpec((1, _R), rep),
        ],
        out_specs=pl.BlockSpec((1, _BBLK), lambda i: (0, i)),
        out_shape=jax.ShapeDtypeStruct((1, B), jnp.float32),
        compiler_params=pltpu.CompilerParams(
            dimension_semantics=("arbitrary",),
        ),
    )(xt, sig, ma, mb, c1c, c1r, c2c, c2r)
    return out.reshape(B)


# speedup vs baseline: 3.4853x; 1.6731x over previous
import jax
import jax.numpy as jnp
from jax.experimental import pallas as pl
from jax.experimental.pallas import tpu as pltpu

_R = 32
_A = 6
_BBLK = 2048


def _rank_mask(c_col, c_row):
    ii = jax.lax.broadcasted_iota(jnp.int32, (_R, _R), 0)
    jj = jax.lax.broadcasted_iota(jnp.int32, (_R, _R), 1)
    cmp = (c_col < c_row) | ((c_col == c_row) & (ii < jj))
    rank = jnp.sum(cmp.astype(jnp.int32), axis=0, keepdims=True)
    pre = (rank <= ii).astype(jnp.float32)
    return pre, 1.0 - pre


def _body(xt_ref, sig_ref, ma_ref, mb_ref,
          c1c_ref, c1r_ref, c2c_ref, c2r_ref, out_ref):
    sig = sig_ref[...] + 0.0001
    m1 = jnp.minimum(ma_ref[...], mb_ref[...])
    m2 = jnp.maximum(ma_ref[...], mb_ref[...])
    ninv = -1.0 / (2.0 * sig * sig)

    eU = jnp.zeros((_R, _BBLK), jnp.float32)
    eL = jnp.zeros((_R, _BBLK), jnp.float32)
    for a in range(_A):
        xa = xt_ref[a:a + 1, :]
        m1a = m1[:, a:a + 1]
        m2a = m2[:, a:a + 1]
        ninva = ninv[:, a:a + 1]
        d1 = xa - m1a
        d2 = xa - m2a
        e1 = (d1 * d1) * ninva
        e2 = (d2 * d2) * ninva
        up = jnp.where(d1 * d2 <= 0.0, 0.0, jnp.maximum(e1, e2))
        eU = eU + up
        eL = eL + jnp.minimum(e1, e2)
    emax = jnp.max(eU, axis=0, keepdims=True)
    UU = jnp.exp(eU - emax)
    LL = jnp.exp(eL - emax)

    c1c = c1c_ref[...]
    c2c = c2c_ref[...]
    M1p, M1s = _rank_mask(c1c, c1r_ref[...])
    M2p, M2s = _rank_mask(c2c, c2r_ref[...])

    def _dot(m, v):
        return jnp.dot(m, v, preferred_element_type=jnp.float32)

    c1U = c1c * UU
    c1L = c1c * LL
    s0 = jnp.sum(c1L, axis=0, keepdims=True)
    s10 = jnp.sum(LL, axis=0, keepdims=True)
    s = _dot(M1p, c1U) + _dot(M1s, c1L)
    s1 = _dot(M1p, UU) + _dot(M1s, LL)
    left = jnp.minimum(s0 / s10, jnp.min(s / s1, axis=0, keepdims=True))

    c2U = c2c * UU
    c2L = c2c * LL
    t0 = jnp.sum(c2U, axis=0, keepdims=True)
    t10 = jnp.sum(UU, axis=0, keepdims=True)
    t = _dot(M2p, c2L) + _dot(M2s, c2U)
    t1 = _dot(M2p, LL) + _dot(M2s, UU)
    right = jnp.maximum(t0 / t10, jnp.max(t / t1, axis=0, keepdims=True))

    out_ref[...] = (left + right) * 0.5


def kernel(input_data, FRB_weights, c1, c2):
    B = input_data.shape[0]
    xt = input_data.T
    sig = FRB_weights[0:_R * _A].reshape(_R, _A)
    ma = FRB_weights[1:_R * _A + 1].reshape(_R, _A)
    mb = FRB_weights[2:_R * _A + 2].reshape(_R, _A)
    c1c = c1.reshape(_R, 1)
    c1r = c1.reshape(1, _R)
    c2c = c2.reshape(_R, 1)
    c2r = c2.reshape(1, _R)

    grid = (B // _BBLK,)
    rep = lambda i: (0, 0)
    out = pl.pallas_call(
        _body,
        grid=grid,
        in_specs=[
            pl.BlockSpec((_A, _BBLK), lambda i: (0, i)),
            pl.BlockSpec((_R, _A), rep),
            pl.BlockSpec((_R, _A), rep),
            pl.BlockSpec((_R, _A), rep),
            pl.BlockSpec((_R, 1), rep),
            pl.BlockSpec((1, _R), rep),
            pl.BlockSpec((_R, 1), rep),
            pl.BlockSpec((1, _R), rep),
        ],
        out_specs=pl.BlockSpec((1, _BBLK), lambda i: (0, i)),
        out_shape=jax.ShapeDtypeStruct((1, B), jnp.float32),
        compiler_params=pltpu.CompilerParams(
            dimension_semantics=("arbitrary",),
        ),
    )(xt, sig, ma, mb, c1c, c1r, c2c, c2r)
    return out.reshape(B)


# speedup vs baseline: 3.7629x; 1.0796x over previous
import jax
import jax.numpy as jnp
from jax.experimental import pallas as pl
from jax.experimental.pallas import tpu as pltpu

_R = 32
_A = 6
_BBLK = 4096


def _rank_mask(c_col, c_row):
    ii = jax.lax.broadcasted_iota(jnp.int32, (_R, _R), 0)
    jj = jax.lax.broadcasted_iota(jnp.int32, (_R, _R), 1)
    cmp = (c_col < c_row) | ((c_col == c_row) & (ii < jj))
    rank = jnp.sum(cmp.astype(jnp.int32), axis=0, keepdims=True)
    pre = (rank <= ii).astype(jnp.float32)
    return pre, 1.0 - pre


def _body(xt_ref, sig_ref, ma_ref, mb_ref,
          c1c_ref, c1r_ref, c2c_ref, c2r_ref, out_ref):
    sig = sig_ref[...] + 0.0001
    m1 = jnp.minimum(ma_ref[...], mb_ref[...])
    m2 = jnp.maximum(ma_ref[...], mb_ref[...])
    ninv = -1.0 / (2.0 * sig * sig)

    eU = jnp.zeros((_R, _BBLK), jnp.float32)
    eL = jnp.zeros((_R, _BBLK), jnp.float32)
    for a in range(_A):
        xa = xt_ref[a:a + 1, :]
        m1a = m1[:, a:a + 1]
        m2a = m2[:, a:a + 1]
        ninva = ninv[:, a:a + 1]
        d1 = xa - m1a
        d2 = xa - m2a
        e1 = (d1 * d1) * ninva
        e2 = (d2 * d2) * ninva
        up = jnp.where(d1 * d2 <= 0.0, 0.0, jnp.maximum(e1, e2))
        eU = eU + up
        eL = eL + jnp.minimum(e1, e2)
    emax = jnp.max(eU, axis=0, keepdims=True)
    UU = jnp.exp(eU - emax)
    LL = jnp.exp(eL - emax)

    c1c = c1c_ref[...]
    c2c = c2c_ref[...]
    M1p, M1s = _rank_mask(c1c, c1r_ref[...])
    M2p, M2s = _rank_mask(c2c, c2r_ref[...])

    def _dot(m, v):
        return jnp.dot(m, v, preferred_element_type=jnp.float32)

    c1U = c1c * UU
    c1L = c1c * LL
    s0 = jnp.sum(c1L, axis=0, keepdims=True)
    s10 = jnp.sum(LL, axis=0, keepdims=True)
    s = _dot(M1p, c1U) + _dot(M1s, c1L)
    s1 = _dot(M1p, UU) + _dot(M1s, LL)
    left = jnp.minimum(s0 / s10, jnp.min(s / s1, axis=0, keepdims=True))

    c2U = c2c * UU
    c2L = c2c * LL
    t0 = jnp.sum(c2U, axis=0, keepdims=True)
    t10 = jnp.sum(UU, axis=0, keepdims=True)
    t = _dot(M2p, c2L) + _dot(M2s, c2U)
    t1 = _dot(M2p, LL) + _dot(M2s, UU)
    right = jnp.maximum(t0 / t10, jnp.max(t / t1, axis=0, keepdims=True))

    out_ref[...] = (left + right) * 0.5


def kernel(input_data, FRB_weights, c1, c2):
    B = input_data.shape[0]
    xt = input_data.T
    sig = FRB_weights[0:_R * _A].reshape(_R, _A)
    ma = FRB_weights[1:_R * _A + 1].reshape(_R, _A)
    mb = FRB_weights[2:_R * _A + 2].reshape(_R, _A)
    c1c = c1.reshape(_R, 1)
    c1r = c1.reshape(1, _R)
    c2c = c2.reshape(_R, 1)
    c2r = c2.reshape(1, _R)

    grid = (B // _BBLK,)
    rep = lambda i: (0, 0)
    out = pl.pallas_call(
        _body,
        grid=grid,
        in_specs=[
            pl.BlockSpec((_A, _BBLK), lambda i: (0, i)),
            pl.BlockSpec((_R, _A), rep),
            pl.BlockSpec((_R, _A), rep),
            pl.BlockSpec((_R, _A), rep),
            pl.BlockSpec((_R, 1), rep),
            pl.BlockSpec((1, _R), rep),
            pl.BlockSpec((_R, 1), rep),
            pl.BlockSpec((1, _R), rep),
        ],
        out_specs=pl.BlockSpec((1, _BBLK), lambda i: (0, i)),
        out_shape=jax.ShapeDtypeStruct((1, B), jnp.float32),
        compiler_params=pltpu.CompilerParams(
            dimension_semantics=("arbitrary",),
        ),
    )(xt, sig, ma, mb, c1c, c1r, c2c, c2r)
    return out.reshape(B)


# speedup vs baseline: 3.8737x; 1.0294x over previous
import jax
import jax.numpy as jnp
from jax.experimental import pallas as pl
from jax.experimental.pallas import tpu as pltpu

_R = 32
_A = 6
_BBLK = 8192


def _rank_mask(c_col, c_row):
    ii = jax.lax.broadcasted_iota(jnp.int32, (_R, _R), 0)
    jj = jax.lax.broadcasted_iota(jnp.int32, (_R, _R), 1)
    cmp = (c_col < c_row) | ((c_col == c_row) & (ii < jj))
    rank = jnp.sum(cmp.astype(jnp.int32), axis=0, keepdims=True)
    pre = (rank <= ii).astype(jnp.float32)
    return pre, 1.0 - pre


def _body(xt_ref, sig_ref, ma_ref, mb_ref,
          c1c_ref, c1r_ref, c2c_ref, c2r_ref, out_ref):
    sig = sig_ref[...] + 0.0001
    m1 = jnp.minimum(ma_ref[...], mb_ref[...])
    m2 = jnp.maximum(ma_ref[...], mb_ref[...])
    ninv = -1.0 / (2.0 * sig * sig)

    eU = jnp.zeros((_R, _BBLK), jnp.float32)
    eL = jnp.zeros((_R, _BBLK), jnp.float32)
    for a in range(_A):
        xa = xt_ref[a:a + 1, :]
        m1a = m1[:, a:a + 1]
        m2a = m2[:, a:a + 1]
        ninva = ninv[:, a:a + 1]
        d1 = xa - m1a
        d2 = xa - m2a
        e1 = (d1 * d1) * ninva
        e2 = (d2 * d2) * ninva
        up = jnp.where(d1 * d2 <= 0.0, 0.0, jnp.maximum(e1, e2))
        eU = eU + up
        eL = eL + jnp.minimum(e1, e2)
    emax = jnp.max(eU, axis=0, keepdims=True)
    UU = jnp.exp(eU - emax)
    LL = jnp.exp(eL - emax)

    c1c = c1c_ref[...]
    c2c = c2c_ref[...]
    M1p, M1s = _rank_mask(c1c, c1r_ref[...])
    M2p, M2s = _rank_mask(c2c, c2r_ref[...])

    def _dot(m, v):
        return jnp.dot(m, v, preferred_element_type=jnp.float32)

    c1U = c1c * UU
    c1L = c1c * LL
    s0 = jnp.sum(c1L, axis=0, keepdims=True)
    s10 = jnp.sum(LL, axis=0, keepdims=True)
    s = _dot(M1p, c1U) + _dot(M1s, c1L)
    s1 = _dot(M1p, UU) + _dot(M1s, LL)
    left = jnp.minimum(s0 / s10, jnp.min(s / s1, axis=0, keepdims=True))

    c2U = c2c * UU
    c2L = c2c * LL
    t0 = jnp.sum(c2U, axis=0, keepdims=True)
    t10 = jnp.sum(UU, axis=0, keepdims=True)
    t = _dot(M2p, c2L) + _dot(M2s, c2U)
    t1 = _dot(M2p, LL) + _dot(M2s, UU)
    right = jnp.maximum(t0 / t10, jnp.max(t / t1, axis=0, keepdims=True))

    out_ref[...] = (left + right) * 0.5


def kernel(input_data, FRB_weights, c1, c2):
    B = input_data.shape[0]
    xt = input_data.T
    sig = FRB_weights[0:_R * _A].reshape(_R, _A)
    ma = FRB_weights[1:_R * _A + 1].reshape(_R, _A)
    mb = FRB_weights[2:_R * _A + 2].reshape(_R, _A)
    c1c = c1.reshape(_R, 1)
    c1r = c1.reshape(1, _R)
    c2c = c2.reshape(_R, 1)
    c2r = c2.reshape(1, _R)

    grid = (B // _BBLK,)
    rep = lambda i: (0, 0)
    out = pl.pallas_call(
        _body,
        grid=grid,
        in_specs=[
            pl.BlockSpec((_A, _BBLK), lambda i: (0, i)),
            pl.BlockSpec((_R, _A), rep),
            pl.BlockSpec((_R, _A), rep),
            pl.BlockSpec((_R, _A), rep),
            pl.BlockSpec((_R, 1), rep),
            pl.BlockSpec((1, _R), rep),
            pl.BlockSpec((_R, 1), rep),
            pl.BlockSpec((1, _R), rep),
        ],
        out_specs=pl.BlockSpec((1, _BBLK), lambda i: (0, i)),
        out_shape=jax.ShapeDtypeStruct((1, B), jnp.float32),
        compiler_params=pltpu.CompilerParams(
            dimension_semantics=("arbitrary",),
        ),
    )(xt, sig, ma, mb, c1c, c1r, c2c, c2r)
    return out.reshape(B)
